# baseline (speedup 1.0000x reference)
import jax
import jax.numpy as jnp
import numpy as np
from jax import lax
from jax.experimental import pallas as pl
from jax.experimental.pallas import tpu as pltpu, tpu_sc as plsc
import functools

N_NODES = 10000
N_EDGES = 320000
RBF_DIM = 16
DIM = 128

N_CORES = 2
N_SUBCORES = 16
CHUNK = 64
N_CHUNKS = N_EDGES // CHUNK
CA = 108
CB = 212
N_CHUNKS_PAD = N_SUBCORES * (CA + CB)
E_PAD = N_CHUNKS_PAD * CHUNK
ACC_ROWS = 10112
DUMMY_DST = N_NODES

TC_BLK = 2000
TC_GRID = N_EDGES // TC_BLK


def _softplus(x, beta=0.5, threshold=14.0):
    return jnp.where(beta * x > threshold, x,
                     (1.0 / beta) * jnp.log1p(jnp.exp(beta * x)))


def _edge_mlp_body(rbf_ref, w1_ref, b1_ref, w2_ref, b2_ref, out_ref):
    x = rbf_ref[:]
    h = jnp.dot(x, w1_ref[:], preferred_element_type=jnp.float32) + b1_ref[:]
    h = _softplus(h)
    out_ref[:] = jnp.dot(h, w2_ref[:],
                         preferred_element_type=jnp.float32) + b2_ref[:]


def _edge_mlp(rbf, W1b, b1, W2b, b2):
    return pl.pallas_call(
        _edge_mlp_body,
        grid=(TC_GRID,),
        in_specs=[
            pl.BlockSpec((TC_BLK, RBF_DIM), lambda g: (g, 0)),
            pl.BlockSpec((RBF_DIM, DIM), lambda g: (0, 0)),
            pl.BlockSpec((1, DIM), lambda g: (0, 0)),
            pl.BlockSpec((DIM, DIM), lambda g: (0, 0)),
            pl.BlockSpec((1, DIM), lambda g: (0, 0)),
        ],
        out_specs=pl.BlockSpec((TC_BLK, DIM), lambda g: (g, 0)),
        out_shape=jax.ShapeDtypeStruct((N_EDGES, DIM), jnp.float32),
    )(rbf, W1b, b1.reshape(1, DIM), W2b, b2.reshape(1, DIM))


def _combine_body(p_ref, o_ref):
    o_ref[:] = p_ref[0] + p_ref[1]


def _combine(partials):
    blk = 2000
    return pl.pallas_call(
        _combine_body,
        grid=(N_NODES // blk,),
        in_specs=[pl.BlockSpec((2, blk, DIM), lambda g: (0, g, 0))],
        out_specs=pl.BlockSpec((blk, DIM), lambda g: (g, 0)),
        out_shape=jax.ShapeDtypeStruct((N_NODES, DIM), jnp.float32),
    )(partials)


@functools.partial(
    pl.kernel,
    out_type=jax.ShapeDtypeStruct((N_CORES, ACC_ROWS, DIM), jnp.float32),
    mesh=plsc.VectorSubcoreMesh(core_axis_name="c", subcore_axis_name="s"),
    scratch_types=[
        [pltpu.VMEM((2, CHUNK), jnp.int32)] * 2,
        [pltpu.VMEM((CHUNK,), jnp.int32)] * 2,
        [pltpu.VMEM((CHUNK, DIM), jnp.float32)] * 2,
        [pltpu.VMEM((CHUNK, DIM), jnp.float32)] * 2,
        [pltpu.VMEM((CHUNK, DIM), jnp.float32)] * 2,
        pltpu.VMEM_SHARED((ACC_ROWS, DIM), jnp.float32),
        [pltpu.SemaphoreType.DMA] * 2,
        [pltpu.SemaphoreType.DMA] * 2,
        [pltpu.SemaphoreType.DMA] * 2,
        [pltpu.SemaphoreType.DMA] * 2,
    ],
)
def _sc_scatter(node_feat, h, idx2_h, out_h,
                idx, dsts, rows, hbuf, msg, acc, isem, gsem, hsem, ssem):
    cid = lax.axis_index("c")
    sid = lax.axis_index("s")
    n_tile_chunks = jnp.where(cid == 0, CA, CB)
    crow = cid * (N_SUBCORES * CA) + sid * n_tile_chunks

    zeros = jnp.zeros((16,), jnp.float32)

    def zero_row(i, carry):
        for j in range(DIM // 16):
            msg[0][i, pl.ds(j * 16, 16)] = zeros
        return carry
    lax.fori_loop(0, CHUNK, zero_row, 0)
    rows_per_tile = ACC_ROWS // N_SUBCORES
    for k in range(10):
        r = sid * rows_per_tile + k * CHUNK
        n = CHUNK if k < 9 else rows_per_tile - 9 * CHUNK
        pltpu.sync_copy(msg[0].at[pl.ds(0, n)], acc.at[pl.ds(r, n)])
    plsc.subcore_barrier()

    def start_idx(g, q):
        pltpu.async_copy(idx2_h.at[crow + g], idx[q], isem[q])

    def start_in(g, b):
        row = crow + g
        hrow = jnp.where(row < N_CHUNKS, row, 0) * CHUNK
        pltpu.async_copy(node_feat.at[idx[b].at[0]], rows[b], gsem[b])
        pltpu.async_copy(h.at[pl.ds(hrow, CHUNK)], hbuf[b], hsem[b])

    start_idx(0, 0)
    start_idx(1, 1)
    pltpu.make_async_copy(idx2_h.at[crow], idx[0], isem[0]).wait()
    start_in(0, 0)

    n_outer = n_tile_chunks // 2
    def outer(t, carry):
        for p in range(2):
            g = 2 * t + p
            b, nb = p, 1 - p
            pltpu.make_async_copy(node_feat.at[idx[b].at[0]], rows[b],
                                  gsem[b]).wait()
            pltpu.make_async_copy(h.at[pl.ds(0, CHUNK)], hbuf[b],
                                  hsem[b]).wait()

            @pl.when(t > 0)
            def _():
                pltpu.make_async_copy(msg[b], acc.at[dsts[b]],
                                      ssem[b]).wait()

            for k in range(CHUNK // 16):
                dsts[b][pl.ds(k * 16, 16)] = idx[b][1, pl.ds(k * 16, 16)]

            @pl.when(t < n_outer - 1)
            def _():
                start_idx(g + 2, b)

            if p == 0:
                pltpu.make_async_copy(idx2_h.at[crow], idx[nb],
                                      isem[nb]).wait()
                start_in(g + 1, nb)
            else:
                @pl.when(t < n_outer - 1)
                def _():
                    pltpu.make_async_copy(idx2_h.at[crow], idx[nb],
                                          isem[nb]).wait()
                    start_in(g + 1, nb)

            def mul_row(i, c2):
                for j in range(DIM // 16):
                    s = pl.ds(j * 16, 16)
                    msg[b][i, s] = rows[b][i, s] * hbuf[b][i, s]
                return c2
            lax.fori_loop(0, CHUNK, mul_row, 0)

            pltpu.async_copy(msg[b], acc.at[dsts[b]], ssem[b], add=True)
        return carry
    lax.fori_loop(0, n_outer, outer, 0)

    for b in range(2):
        pltpu.make_async_copy(msg[b], acc.at[dsts[b]], ssem[b]).wait()

    plsc.subcore_barrier()

    for k in range(10):
        r = sid * rows_per_tile + k * CHUNK
        n = CHUNK if k < 9 else rows_per_tile - 9 * CHUNK
        pltpu.sync_copy(acc.at[pl.ds(r, n)], msg[0].at[pl.ds(0, n)])
        pltpu.sync_copy(msg[0].at[pl.ds(0, n)], out_h.at[cid].at[pl.ds(r, n)])


def kernel(node_feat, rbf, edge_index, W1, b1, W2, b2):
    pad = E_PAD - N_EDGES
    src = jnp.concatenate(
        [edge_index[0].astype(jnp.int32), jnp.zeros((pad,), jnp.int32)])
    dst = jnp.concatenate(
        [edge_index[1].astype(jnp.int32),
         DUMMY_DST + (jnp.arange(pad, dtype=jnp.int32)
                      % (ACC_ROWS - N_NODES))])
    idx2 = jnp.stack([src.reshape(N_CHUNKS_PAD, CHUNK),
                      dst.reshape(N_CHUNKS_PAD, CHUNK)], axis=1)
    h = _edge_mlp(rbf, W1, b1, W2, b2)
    partials = _sc_scatter(node_feat, h, idx2)
    return _combine(partials)


# speedup vs baseline: 1.0793x; 1.0793x over previous
import jax
import jax.numpy as jnp
import numpy as np
from jax import lax
from jax.experimental import pallas as pl
from jax.experimental.pallas import tpu as pltpu, tpu_sc as plsc
import functools

N_NODES = 10000
N_EDGES = 320000
RBF_DIM = 16
DIM = 128

N_CORES = 2
N_SUBCORES = 16
CHUNK = 64
N_CHUNKS = N_EDGES // CHUNK
CA = 212
CB = 108
N_CHUNKS_PAD = N_SUBCORES * (CA + CB)
E_PAD = N_CHUNKS_PAD * CHUNK
ACC_ROWS = 10112
DUMMY_DST = N_NODES

TC_BLK = 2000
TC_GRID = N_EDGES // TC_BLK


def _softplus(x, beta=0.5, threshold=14.0):
    return jnp.where(beta * x > threshold, x,
                     (1.0 / beta) * jnp.log1p(jnp.exp(beta * x)))


def _edge_mlp_body(rbf_ref, w1_ref, b1_ref, w2_ref, b2_ref, out_ref):
    x = rbf_ref[:]
    h = jnp.dot(x, w1_ref[:], preferred_element_type=jnp.float32) + b1_ref[:]
    h = _softplus(h)
    out_ref[:] = jnp.dot(h, w2_ref[:],
                         preferred_element_type=jnp.float32) + b2_ref[:]


def _edge_mlp(rbf, W1b, b1, W2b, b2):
    return pl.pallas_call(
        _edge_mlp_body,
        grid=(TC_GRID,),
        in_specs=[
            pl.BlockSpec((TC_BLK, RBF_DIM), lambda g: (g, 0)),
            pl.BlockSpec((RBF_DIM, DIM), lambda g: (0, 0)),
            pl.BlockSpec((1, DIM), lambda g: (0, 0)),
            pl.BlockSpec((DIM, DIM), lambda g: (0, 0)),
            pl.BlockSpec((1, DIM), lambda g: (0, 0)),
        ],
        out_specs=pl.BlockSpec((TC_BLK, DIM), lambda g: (g, 0)),
        out_shape=jax.ShapeDtypeStruct((N_EDGES, DIM), jnp.float32),
    )(rbf, W1b, b1.reshape(1, DIM), W2b, b2.reshape(1, DIM))


def _combine_body(p_ref, o_ref):
    o_ref[:] = p_ref[0] + p_ref[1]


def _combine(partials):
    blk = 2000
    return pl.pallas_call(
        _combine_body,
        grid=(N_NODES // blk,),
        in_specs=[pl.BlockSpec((2, blk, DIM), lambda g: (0, g, 0))],
        out_specs=pl.BlockSpec((blk, DIM), lambda g: (g, 0)),
        out_shape=jax.ShapeDtypeStruct((N_NODES, DIM), jnp.float32),
    )(partials)


@functools.partial(
    pl.kernel,
    out_type=jax.ShapeDtypeStruct((N_CORES, ACC_ROWS, DIM), jnp.float32),
    mesh=plsc.VectorSubcoreMesh(core_axis_name="c", subcore_axis_name="s"),
    scratch_types=[
        [pltpu.VMEM((2, CHUNK), jnp.int32)] * 2,
        [pltpu.VMEM((CHUNK,), jnp.int32)] * 2,
        [pltpu.VMEM((CHUNK, DIM), jnp.float32)] * 2,
        [pltpu.VMEM((CHUNK, DIM), jnp.float32)] * 2,
        [pltpu.VMEM((CHUNK, DIM), jnp.float32)] * 2,
        pltpu.VMEM_SHARED((ACC_ROWS, DIM), jnp.float32),
        [pltpu.SemaphoreType.DMA] * 2,
        [pltpu.SemaphoreType.DMA] * 2,
        [pltpu.SemaphoreType.DMA] * 2,
        [pltpu.SemaphoreType.DMA] * 2,
    ],
)
def _sc_scatter(node_feat, h, idx2_h, out_h,
                idx, dsts, rows, hbuf, msg, acc, isem, gsem, hsem, ssem):
    cid = lax.axis_index("c")
    sid = lax.axis_index("s")
    n_tile_chunks = jnp.where(cid == 0, CA, CB)
    crow = cid * (N_SUBCORES * CA) + sid * n_tile_chunks

    zeros = jnp.zeros((16,), jnp.float32)

    def zero_row(i, carry):
        for j in range(DIM // 16):
            msg[0][i, pl.ds(j * 16, 16)] = zeros
        return carry
    lax.fori_loop(0, CHUNK, zero_row, 0)
    rows_per_tile = ACC_ROWS // N_SUBCORES
    for k in range(10):
        r = sid * rows_per_tile + k * CHUNK
        n = CHUNK if k < 9 else rows_per_tile - 9 * CHUNK
        pltpu.sync_copy(msg[0].at[pl.ds(0, n)], acc.at[pl.ds(r, n)])
    plsc.subcore_barrier()

    def start_idx(g, q):
        pltpu.async_copy(idx2_h.at[crow + g], idx[q], isem[q])

    def start_in(g, b):
        row = crow + g
        hrow = jnp.where(row < N_CHUNKS, row, 0) * CHUNK
        pltpu.async_copy(node_feat.at[idx[b].at[0]], rows[b], gsem[b])
        pltpu.async_copy(h.at[pl.ds(hrow, CHUNK)], hbuf[b], hsem[b])

    start_idx(0, 0)
    start_idx(1, 1)
    pltpu.make_async_copy(idx2_h.at[crow], idx[0], isem[0]).wait()
    start_in(0, 0)

    n_outer = n_tile_chunks // 2
    def outer(t, carry):
        for p in range(2):
            g = 2 * t + p
            b, nb = p, 1 - p
            pltpu.make_async_copy(node_feat.at[idx[b].at[0]], rows[b],
                                  gsem[b]).wait()
            pltpu.make_async_copy(h.at[pl.ds(0, CHUNK)], hbuf[b],
                                  hsem[b]).wait()

            @pl.when(t > 0)
            def _():
                pltpu.make_async_copy(msg[b], acc.at[dsts[b]],
                                      ssem[b]).wait()

            for k in range(CHUNK // 16):
                dsts[b][pl.ds(k * 16, 16)] = idx[b][1, pl.ds(k * 16, 16)]

            @pl.when(t < n_outer - 1)
            def _():
                start_idx(g + 2, b)

            if p == 0:
                pltpu.make_async_copy(idx2_h.at[crow], idx[nb],
                                      isem[nb]).wait()
                start_in(g + 1, nb)
            else:
                @pl.when(t < n_outer - 1)
                def _():
                    pltpu.make_async_copy(idx2_h.at[crow], idx[nb],
                                          isem[nb]).wait()
                    start_in(g + 1, nb)

            def mul_row(i, c2):
                for j in range(DIM // 16):
                    s = pl.ds(j * 16, 16)
                    msg[b][i, s] = rows[b][i, s] * hbuf[b][i, s]
                return c2
            lax.fori_loop(0, CHUNK, mul_row, 0)

            pltpu.async_copy(msg[b], acc.at[dsts[b]], ssem[b], add=True)
        return carry
    lax.fori_loop(0, n_outer, outer, 0)

    for b in range(2):
        pltpu.make_async_copy(msg[b], acc.at[dsts[b]], ssem[b]).wait()

    plsc.subcore_barrier()

    for k in range(10):
        r = sid * rows_per_tile + k * CHUNK
        n = CHUNK if k < 9 else rows_per_tile - 9 * CHUNK
        pltpu.sync_copy(acc.at[pl.ds(r, n)], msg[0].at[pl.ds(0, n)])
        pltpu.sync_copy(msg[0].at[pl.ds(0, n)], out_h.at[cid].at[pl.ds(r, n)])


def kernel(node_feat, rbf, edge_index, W1, b1, W2, b2):
    pad = E_PAD - N_EDGES
    src = jnp.concatenate(
        [edge_index[0].astype(jnp.int32), jnp.zeros((pad,), jnp.int32)])
    dst = jnp.concatenate(
        [edge_index[1].astype(jnp.int32),
         DUMMY_DST + (jnp.arange(pad, dtype=jnp.int32)
                      % (ACC_ROWS - N_NODES))])
    idx2 = jnp.stack([src.reshape(N_CHUNKS_PAD, CHUNK),
                      dst.reshape(N_CHUNKS_PAD, CHUNK)], axis=1)
    h = _edge_mlp(rbf, W1, b1, W2, b2)
    partials = _sc_scatter(node_feat, h, idx2)
    return _combine(partials)


# speedup vs baseline: 1.1507x; 1.0661x over previous
import jax
import jax.numpy as jnp
import numpy as np
from jax import lax
from jax.experimental import pallas as pl
from jax.experimental.pallas import tpu as pltpu, tpu_sc as plsc
import functools

N_NODES = 10000
N_EDGES = 320000
RBF_DIM = 16
DIM = 128

N_CORES = 2
N_SUBCORES = 16
CHUNK = 64
N_CHUNKS = N_EDGES // CHUNK
CA = 256
CB = 64
N_CHUNKS_PAD = N_SUBCORES * (CA + CB)
E_PAD = N_CHUNKS_PAD * CHUNK
ACC_ROWS = 10112
DUMMY_DST = N_NODES

TC_BLK = 2000
TC_GRID = N_EDGES // TC_BLK


def _softplus(x, beta=0.5, threshold=14.0):
    return jnp.where(beta * x > threshold, x,
                     (1.0 / beta) * jnp.log1p(jnp.exp(beta * x)))


def _edge_mlp_body(rbf_ref, w1_ref, b1_ref, w2_ref, b2_ref, out_ref):
    x = rbf_ref[:]
    h = jnp.dot(x, w1_ref[:], preferred_element_type=jnp.float32) + b1_ref[:]
    h = _softplus(h).astype(jnp.bfloat16)
    out_ref[:] = jnp.dot(h, w2_ref[:],
                         preferred_element_type=jnp.float32) + b2_ref[:]


def _edge_mlp(rbf, W1b, b1, W2b, b2):
    return pl.pallas_call(
        _edge_mlp_body,
        grid=(TC_GRID,),
        in_specs=[
            pl.BlockSpec((TC_BLK, RBF_DIM), lambda g: (g, 0)),
            pl.BlockSpec((RBF_DIM, DIM), lambda g: (0, 0)),
            pl.BlockSpec((1, DIM), lambda g: (0, 0)),
            pl.BlockSpec((DIM, DIM), lambda g: (0, 0)),
            pl.BlockSpec((1, DIM), lambda g: (0, 0)),
        ],
        out_specs=pl.BlockSpec((TC_BLK, DIM), lambda g: (g, 0)),
        out_shape=jax.ShapeDtypeStruct((N_EDGES, DIM), jnp.float32),
    )(rbf, W1b, b1.reshape(1, DIM), W2b, b2.reshape(1, DIM))


def _combine_body(p_ref, o_ref):
    o_ref[:] = p_ref[0] + p_ref[1]


def _combine(partials):
    blk = 2000
    return pl.pallas_call(
        _combine_body,
        grid=(N_NODES // blk,),
        in_specs=[pl.BlockSpec((2, blk, DIM), lambda g: (0, g, 0))],
        out_specs=pl.BlockSpec((blk, DIM), lambda g: (g, 0)),
        out_shape=jax.ShapeDtypeStruct((N_NODES, DIM), jnp.float32),
    )(partials)


@functools.partial(
    pl.kernel,
    out_type=jax.ShapeDtypeStruct((N_CORES, ACC_ROWS, DIM), jnp.float32),
    mesh=plsc.VectorSubcoreMesh(core_axis_name="c", subcore_axis_name="s"),
    scratch_types=[
        [pltpu.VMEM((2, CHUNK), jnp.int32)] * 2,
        [pltpu.VMEM((CHUNK,), jnp.int32)] * 2,
        [pltpu.VMEM((CHUNK, DIM), jnp.float32)] * 2,
        [pltpu.VMEM((CHUNK, DIM), jnp.float32)] * 2,
        [pltpu.VMEM((CHUNK, DIM), jnp.float32)] * 2,
        pltpu.VMEM_SHARED((ACC_ROWS, DIM), jnp.float32),
        [pltpu.SemaphoreType.DMA] * 2,
        [pltpu.SemaphoreType.DMA] * 2,
        [pltpu.SemaphoreType.DMA] * 2,
        [pltpu.SemaphoreType.DMA] * 2,
    ],
)
def _sc_scatter(node_feat, h, idx2_h, out_h,
                idx, dsts, rows, hbuf, msg, acc, isem, gsem, hsem, ssem):
    cid = lax.axis_index("c")
    sid = lax.axis_index("s")
    n_tile_chunks = jnp.where(cid == 0, CA, CB)
    crow = cid * (N_SUBCORES * CA) + sid * n_tile_chunks

    zeros = jnp.zeros((16,), jnp.float32)

    def zero_row(i, carry):
        for j in range(DIM // 16):
            msg[0][i, pl.ds(j * 16, 16)] = zeros
        return carry
    lax.fori_loop(0, CHUNK, zero_row, 0)
    rows_per_tile = ACC_ROWS // N_SUBCORES
    for k in range(10):
        r = sid * rows_per_tile + k * CHUNK
        n = CHUNK if k < 9 else rows_per_tile - 9 * CHUNK
        pltpu.sync_copy(msg[0].at[pl.ds(0, n)], acc.at[pl.ds(r, n)])
    plsc.subcore_barrier()

    def start_idx(g, q):
        pltpu.async_copy(idx2_h.at[crow + g], idx[q], isem[q])

    def start_in(g, b):
        row = crow + g
        hrow = jnp.where(row < N_CHUNKS, row, 0) * CHUNK
        pltpu.async_copy(node_feat.at[idx[b].at[0]], rows[b], gsem[b])
        pltpu.async_copy(h.at[pl.ds(hrow, CHUNK)], hbuf[b], hsem[b])

    start_idx(0, 0)
    start_idx(1, 1)
    pltpu.make_async_copy(idx2_h.at[crow], idx[0], isem[0]).wait()
    start_in(0, 0)

    n_outer = n_tile_chunks // 2
    def outer(t, carry):
        for p in range(2):
            g = 2 * t + p
            b, nb = p, 1 - p
            pltpu.make_async_copy(node_feat.at[idx[b].at[0]], rows[b],
                                  gsem[b]).wait()
            pltpu.make_async_copy(h.at[pl.ds(0, CHUNK)], hbuf[b],
                                  hsem[b]).wait()

            @pl.when(t > 0)
            def _():
                pltpu.make_async_copy(msg[b], acc.at[dsts[b]],
                                      ssem[b]).wait()

            for k in range(CHUNK // 16):
                dsts[b][pl.ds(k * 16, 16)] = idx[b][1, pl.ds(k * 16, 16)]

            @pl.when(t < n_outer - 1)
            def _():
                start_idx(g + 2, b)

            if p == 0:
                pltpu.make_async_copy(idx2_h.at[crow], idx[nb],
                                      isem[nb]).wait()
                start_in(g + 1, nb)
            else:
                @pl.when(t < n_outer - 1)
                def _():
                    pltpu.make_async_copy(idx2_h.at[crow], idx[nb],
                                          isem[nb]).wait()
                    start_in(g + 1, nb)

            def mul_row(i, c2):
                for j in range(DIM // 16):
                    s = pl.ds(j * 16, 16)
                    msg[b][i, s] = rows[b][i, s] * hbuf[b][i, s]
                return c2
            lax.fori_loop(0, CHUNK, mul_row, 0)

            pltpu.async_copy(msg[b], acc.at[dsts[b]], ssem[b], add=True)
        return carry
    lax.fori_loop(0, n_outer, outer, 0)

    for b in range(2):
        pltpu.make_async_copy(msg[b], acc.at[dsts[b]], ssem[b]).wait()

    plsc.subcore_barrier()

    for k in range(10):
        r = sid * rows_per_tile + k * CHUNK
        n = CHUNK if k < 9 else rows_per_tile - 9 * CHUNK
        pltpu.sync_copy(acc.at[pl.ds(r, n)], msg[0].at[pl.ds(0, n)])
        pltpu.sync_copy(msg[0].at[pl.ds(0, n)], out_h.at[cid].at[pl.ds(r, n)])


def kernel(node_feat, rbf, edge_index, W1, b1, W2, b2):
    pad = E_PAD - N_EDGES
    src = jnp.concatenate(
        [edge_index[0].astype(jnp.int32), jnp.zeros((pad,), jnp.int32)])
    dst = jnp.concatenate(
        [edge_index[1].astype(jnp.int32),
         DUMMY_DST + (jnp.arange(pad, dtype=jnp.int32)
                      % (ACC_ROWS - N_NODES))])
    idx2 = jnp.stack([src.reshape(N_CHUNKS_PAD, CHUNK),
                      dst.reshape(N_CHUNKS_PAD, CHUNK)], axis=1)
    h = _edge_mlp(rbf.astype(jnp.bfloat16), W1.astype(jnp.bfloat16),
                  b1, W2.astype(jnp.bfloat16), b2)
    partials = _sc_scatter(node_feat, h, idx2)
    return _combine(partials)


# speedup vs baseline: 1.1508x; 1.0001x over previous
import jax
import jax.numpy as jnp
from jax import lax
from jax.experimental import pallas as pl
from jax.experimental.pallas import tpu as pltpu, tpu_sc as plsc
import functools

N_NODES = 10000
N_EDGES = 320000
RBF_DIM = 16
DIM = 128

N_CORES = 2
N_SUBCORES = 16
CHUNK = 64
N_CHUNKS = N_EDGES // CHUNK
CA = 256
CB = 64
N_CHUNKS_PAD = N_SUBCORES * (CA + CB)
E_PAD = N_CHUNKS_PAD * CHUNK
ACC_ROWS = 10112
DUMMY_DST = N_NODES

TC_BLK = 2000
TC_GRID = N_EDGES // TC_BLK


def _softplus(x, beta=0.5, threshold=14.0):
    return jnp.where(beta * x > threshold, x,
                     (1.0 / beta) * jnp.log1p(jnp.exp(beta * x)))


def _edge_mlp_body(rbf_ref, w1_ref, b1_ref, w2_ref, b2_ref, out_ref):
    x = rbf_ref[:]
    h = jnp.dot(x, w1_ref[:], preferred_element_type=jnp.float32) + b1_ref[:]
    h = _softplus(h).astype(jnp.bfloat16)
    out_ref[:] = jnp.dot(h, w2_ref[:],
                         preferred_element_type=jnp.float32) + b2_ref[:]


def _edge_mlp(rbf, W1b, b1, W2b, b2):
    return pl.pallas_call(
        _edge_mlp_body,
        grid=(TC_GRID,),
        in_specs=[
            pl.BlockSpec((TC_BLK, RBF_DIM), lambda g: (g, 0)),
            pl.BlockSpec((RBF_DIM, DIM), lambda g: (0, 0)),
            pl.BlockSpec((1, DIM), lambda g: (0, 0)),
            pl.BlockSpec((DIM, DIM), lambda g: (0, 0)),
            pl.BlockSpec((1, DIM), lambda g: (0, 0)),
        ],
        out_specs=pl.BlockSpec((TC_BLK, DIM), lambda g: (g, 0)),
        out_shape=jax.ShapeDtypeStruct((N_EDGES, DIM), jnp.float32),
    )(rbf, W1b, b1.reshape(1, DIM), W2b, b2.reshape(1, DIM))


def _combine_body(p_ref, o_ref):
    o_ref[:] = p_ref[0] + p_ref[1]


def _combine(partials):
    blk = 2000
    return pl.pallas_call(
        _combine_body,
        grid=(N_NODES // blk,),
        in_specs=[pl.BlockSpec((2, blk, DIM), lambda g: (0, g, 0))],
        out_specs=pl.BlockSpec((blk, DIM), lambda g: (g, 0)),
        out_shape=jax.ShapeDtypeStruct((N_NODES, DIM), jnp.float32),
    )(partials)


@functools.partial(
    pl.kernel,
    out_type=jax.ShapeDtypeStruct((N_CORES, ACC_ROWS, DIM), jnp.float32),
    mesh=plsc.VectorSubcoreMesh(core_axis_name="c", subcore_axis_name="s"),
    scratch_types=[
        [pltpu.VMEM((2, CHUNK), jnp.int32)] * 2,
        [pltpu.VMEM((CHUNK,), jnp.int32)] * 2,
        [pltpu.VMEM((CHUNK, DIM), jnp.float32)] * 2,
        [pltpu.VMEM((CHUNK, DIM), jnp.float32)] * 2,
        [pltpu.VMEM((CHUNK, DIM), jnp.float32)] * 2,
        pltpu.VMEM_SHARED((ACC_ROWS, DIM), jnp.float32),
        [pltpu.SemaphoreType.DMA] * 2,
        [pltpu.SemaphoreType.DMA] * 2,
        [pltpu.SemaphoreType.DMA] * 2,
        [pltpu.SemaphoreType.DMA] * 2,
    ],
)
def _sc_scatter(node_feat, h, idx2_h, out_h,
                idx, dsts, rows, hbuf, msg, acc, isem, gsem, hsem, ssem):
    cid = lax.axis_index("c")
    sid = lax.axis_index("s")
    n_tile_chunks = jnp.where(cid == 0, CA, CB)
    crow = cid * (N_SUBCORES * CA) + sid * n_tile_chunks

    zeros = jnp.zeros((16,), jnp.float32)

    def zero_row(i, carry):
        for j in range(DIM // 16):
            msg[0][i, pl.ds(j * 16, 16)] = zeros
        return carry
    lax.fori_loop(0, CHUNK, zero_row, 0)
    rows_per_tile = ACC_ROWS // N_SUBCORES
    for k in range(10):
        r = sid * rows_per_tile + k * CHUNK
        n = CHUNK if k < 9 else rows_per_tile - 9 * CHUNK
        pltpu.sync_copy(msg[0].at[pl.ds(0, n)], acc.at[pl.ds(r, n)])
    plsc.subcore_barrier()

    def start_idx(g, q):
        pltpu.async_copy(idx2_h.at[crow + g], idx[q], isem[q])

    def start_in(g, b):
        row = crow + g
        hrow = jnp.where(row < N_CHUNKS, row, 0) * CHUNK
        pltpu.async_copy(node_feat.at[idx[b].at[0]], rows[b], gsem[b])
        pltpu.async_copy(h.at[pl.ds(hrow, CHUNK)], hbuf[b], hsem[b])

    start_idx(0, 0)
    start_idx(1, 1)
    pltpu.make_async_copy(idx2_h.at[crow], idx[0], isem[0]).wait()
    start_in(0, 0)

    n_outer = n_tile_chunks // 2
    def outer(t, carry):
        for p in range(2):
            g = 2 * t + p
            b, nb = p, 1 - p
            pltpu.make_async_copy(node_feat.at[idx[b].at[0]], rows[b],
                                  gsem[b]).wait()
            pltpu.make_async_copy(h.at[pl.ds(0, CHUNK)], hbuf[b],
                                  hsem[b]).wait()

            @pl.when(t > 0)
            def _():
                pltpu.make_async_copy(msg[b], acc.at[dsts[b]],
                                      ssem[b]).wait()

            for k in range(CHUNK // 16):
                dsts[b][pl.ds(k * 16, 16)] = idx[b][1, pl.ds(k * 16, 16)]

            @pl.when(t < n_outer - 1)
            def _():
                start_idx(g + 2, b)

            if p == 0:
                pltpu.make_async_copy(idx2_h.at[crow], idx[nb],
                                      isem[nb]).wait()
                start_in(g + 1, nb)
            else:
                @pl.when(t < n_outer - 1)
                def _():
                    pltpu.make_async_copy(idx2_h.at[crow], idx[nb],
                                          isem[nb]).wait()
                    start_in(g + 1, nb)

            def mul_row(i, c2):
                for j in range(DIM // 16):
                    s = pl.ds(j * 16, 16)
                    msg[b][i, s] = rows[b][i, s] * hbuf[b][i, s]
                return c2
            lax.fori_loop(0, CHUNK, mul_row, 0)

            pltpu.async_copy(msg[b], acc.at[dsts[b]], ssem[b], add=True)
        return carry
    lax.fori_loop(0, n_outer, outer, 0)

    for b in range(2):
        pltpu.make_async_copy(msg[b], acc.at[dsts[b]], ssem[b]).wait()

    plsc.subcore_barrier()

    for k in range(10):
        r = sid * rows_per_tile + k * CHUNK
        n = CHUNK if k < 9 else rows_per_tile - 9 * CHUNK
        pltpu.sync_copy(acc.at[pl.ds(r, n)], msg[0].at[pl.ds(0, n)])
        pltpu.sync_copy(msg[0].at[pl.ds(0, n)], out_h.at[cid].at[pl.ds(r, n)])


def kernel(node_feat, rbf, edge_index, W1, b1, W2, b2):
    pad = E_PAD - N_EDGES
    src = jnp.concatenate(
        [edge_index[0].astype(jnp.int32), jnp.zeros((pad,), jnp.int32)])
    dst = jnp.concatenate(
        [edge_index[1].astype(jnp.int32),
         DUMMY_DST + (jnp.arange(pad, dtype=jnp.int32)
                      % (ACC_ROWS - N_NODES))])
    idx2 = jnp.stack([src.reshape(N_CHUNKS_PAD, CHUNK),
                      dst.reshape(N_CHUNKS_PAD, CHUNK)], axis=1)
    h = _edge_mlp(rbf.astype(jnp.bfloat16), W1.astype(jnp.bfloat16),
                  b1, W2.astype(jnp.bfloat16), b2)
    partials = _sc_scatter(node_feat, h, idx2)
    return _combine(partials)
